# baseline (speedup 1.0000x reference)
import jax
import jax.numpy as jnp
from jax.experimental import pallas as pl

IN_DIM = 768
E_DIM = 64
NUM_LEVELS = 3
K = 1024
BETA = 0.25
BATCH = 16384
TILE = 512
HALF = TILE // 2
_CHUNK = 128

_DN = lambda lc, rc: ((lc, rc), ((), ()))


def _dot(a, b, dims=(((1,), (0,)), ((), ()))):
    return jax.lax.dot_general(a, b, dims,
                               precision=jax.lax.Precision.DEFAULT,
                               preferred_element_type=jnp.float32)


def _gather_rows(cbT, idx, n):
    lo = jnp.bitwise_and(idx, _CHUNK - 1)
    hi = jnp.right_shift(idx, 7)
    lo_b = jax.lax.broadcast_in_dim(lo, (E_DIM, n), (1,))
    hi_b = jax.lax.broadcast_in_dim(hi, (E_DIM, n), (1,))
    xqT = jnp.zeros((E_DIM, n), jnp.float32)
    for h in range(K // _CHUNK):
        g = jnp.take_along_axis(cbT[:, h * _CHUNK:(h + 1) * _CHUNK], lo_b,
                                axis=1)
        xqT = jnp.where(hi_b == h, g, xqT)
    return xqT


def _forward(x, ew0, eb0, ew1, eb1, ew2, eb2, cbT_ref, c2_ref,
             dw0, db0, dw1, db1, dw2, db2):
    n = x.shape[0]
    h = jnp.maximum(_dot(x, ew0[...]) + eb0[...], 0.0)
    h = jnp.maximum(_dot(h, ew1[...]) + eb1[...], 0.0)
    resT = _dot(ew2[...], h, _DN((0,), (1,))) + eb2[...]

    kiota = jax.lax.broadcasted_iota(jnp.int32, (K, n), 0)
    xq_accT = jnp.zeros_like(resT)
    loss_sums = []
    idx_rows = []
    for lvl in range(NUM_LEVELS):
        cbT = cbT_ref[lvl]
        d = c2_ref[lvl] - 2.0 * _dot(cbT, resT, _DN((0,), (0,)))
        m = jnp.min(d, axis=0, keepdims=True)
        idx = jnp.min(jnp.where(d == m, kiota, K), axis=0)
        xqT = _gather_rows(cbT, idx, n)
        diffT = xqT - resT
        loss_sums.append(jnp.sum(diffT * diffT))
        xq_accT = xq_accT + xqT
        resT = resT - xqT
        idx_rows.append(idx)

    h = jnp.maximum(_dot(xq_accT, dw0[...], _DN((0,), (0,))) + db0[...], 0.0)
    h = jnp.maximum(_dot(h, dw1[...]) + db1[...], 0.0)
    out = jax.nn.sigmoid(_dot(h, dw2[...]) + db2[...])
    return out, jnp.stack(idx_rows, axis=0), jnp.stack(loss_sums)


def _rqvae_kernel(x_ref, ew0, eb0, ew1, eb1, ew2, eb2, cbT_ref, c2_ref,
                  dw0, db0, dw1, db1, dw2, db2,
                  out_ref, idx_ref, loss_ref):
    i = pl.program_id(0)
    ws = (ew0, eb0, ew1, eb1, ew2, eb2, cbT_ref, c2_ref,
          dw0, db0, dw1, db1, dw2, db2)
    out_a, idx_a, loss_a = _forward(x_ref[:HALF], *ws)
    out_b, idx_b, loss_b = _forward(x_ref[HALF:], *ws)
    out_ref[:HALF] = out_a
    out_ref[HALF:] = out_b
    idx_ref[:, :HALF] = idx_a
    idx_ref[:, HALF:] = idx_b

    @pl.when(i == 0)
    def _():
        loss_ref[...] = jnp.zeros_like(loss_ref)
    loss_ref[...] += (loss_a + loss_b)[None, :]


@jax.jit
def _run(x, enc_W0, enc_b0, enc_W1, enc_b1, enc_W2, enc_b2,
         codebooks, dec_W0, dec_b0, dec_W1, dec_b1, dec_W2, dec_b2):
    grid = BATCH // TILE
    full = lambda shape: pl.BlockSpec(shape, lambda i: (0,) * len(shape))
    cbT = codebooks.transpose(0, 2, 1)
    c2 = jnp.sum(codebooks ** 2, axis=2)[..., None]
    out, idxs, loss = pl.pallas_call(
        _rqvae_kernel,
        grid=(grid,),
        in_specs=[
            pl.BlockSpec((TILE, IN_DIM), lambda i: (i, 0)),
            full(enc_W0.shape), full((1, enc_b0.shape[0])),
            full(enc_W1.shape), full((1, enc_b1.shape[0])),
            full(enc_W2.shape), full((enc_b2.shape[0], 1)),
            full(cbT.shape), full(c2.shape),
            full(dec_W0.shape), full((1, dec_b0.shape[0])),
            full(dec_W1.shape), full((1, dec_b1.shape[0])),
            full(dec_W2.shape), full((1, dec_b2.shape[0])),
        ],
        out_specs=[
            pl.BlockSpec((TILE, IN_DIM), lambda i: (i, 0)),
            pl.BlockSpec((NUM_LEVELS, TILE), lambda i: (0, i)),
            pl.BlockSpec((1, NUM_LEVELS), lambda i: (0, 0)),
        ],
        out_shape=[
            jax.ShapeDtypeStruct((BATCH, IN_DIM), jnp.float32),
            jax.ShapeDtypeStruct((NUM_LEVELS, BATCH), jnp.int32),
            jax.ShapeDtypeStruct((1, NUM_LEVELS), jnp.float32),
        ],
    )(x, enc_W0, enc_b0.reshape(1, -1), enc_W1, enc_b1.reshape(1, -1),
      enc_W2, enc_b2.reshape(-1, 1), cbT, c2,
      dec_W0, dec_b0.reshape(1, -1), dec_W1, dec_b1.reshape(1, -1),
      dec_W2, dec_b2.reshape(1, -1))
    per_level_mse = loss[0] / (BATCH * E_DIM)
    rq_loss = jnp.mean((1.0 + BETA) * per_level_mse)
    return out, rq_loss, idxs.T


def kernel(x, epoch_idx, enc_W0, enc_b0, enc_W1, enc_b1, enc_W2, enc_b2,
           codebooks, dec_W0, dec_b0, dec_W1, dec_b1, dec_W2, dec_b2):
    return _run(x, enc_W0, enc_b0, enc_W1, enc_b1, enc_W2, enc_b2,
                codebooks, dec_W0, dec_b0, dec_W1, dec_b1, dec_W2, dec_b2)


# speedup vs baseline: 1.3130x; 1.3130x over previous
import jax
import jax.numpy as jnp
from jax.experimental import pallas as pl

IN_DIM = 768
E_DIM = 64
NUM_LEVELS = 3
K = 1024
BETA = 0.25
BATCH = 16384
TILE = 512
HALF = TILE // 2
_CHUNK = 128

_DN = lambda lc, rc: ((lc, rc), ((), ()))


def _dot(a, b, dims=(((1,), (0,)), ((), ()))):
    return jax.lax.dot_general(a, b, dims,
                               precision=jax.lax.Precision.DEFAULT,
                               preferred_element_type=jnp.float32)


def _gather_rows(cbT, idx, n):
    lo = jnp.bitwise_and(idx, _CHUNK - 1)
    hi = jnp.right_shift(idx, 7)
    lo_b = jax.lax.broadcast_in_dim(lo, (E_DIM, n), (1,))
    hi_b = jax.lax.broadcast_in_dim(hi, (E_DIM, n), (1,))
    xqT = jnp.zeros((E_DIM, n), jnp.float32)
    for h in range(K // _CHUNK):
        g = jnp.take_along_axis(cbT[:, h * _CHUNK:(h + 1) * _CHUNK], lo_b,
                                axis=1)
        xqT = jnp.where(hi_b == h, g, xqT)
    return xqT


def _forward(x, ew0, eb0, ew1, eb1, ew2, eb2, cbT_ref, c2_ref,
             dw0, db0, dw1, db1, dw2, db2):
    n = x.shape[0]
    h = jnp.maximum(_dot(x, ew0[...]) + eb0[...], 0.0)
    h = jnp.maximum(_dot(h, ew1[...]) + eb1[...], 0.0)
    resT = _dot(ew2[...], h, _DN((0,), (1,))) + eb2[...]

    kiota = jax.lax.broadcasted_iota(jnp.int32, (K, n), 0)
    xq_accT = jnp.zeros_like(resT)
    loss_sums = []
    idx_rows = []
    for lvl in range(NUM_LEVELS):
        cbT = cbT_ref[lvl]
        d = c2_ref[lvl] - 2.0 * _dot(cbT, resT, _DN((0,), (0,)))
        m = jnp.min(d, axis=0, keepdims=True)
        idx = jnp.min(jnp.where(d == m, kiota, K), axis=0)
        xqT = _gather_rows(cbT, idx, n)
        diffT = xqT - resT
        loss_sums.append(jnp.sum(diffT * diffT))
        xq_accT = xq_accT + xqT
        resT = resT - xqT
        idx_rows.append(idx)

    h = jnp.maximum(_dot(xq_accT, dw0[...], _DN((0,), (0,))) + db0[...], 0.0)
    h = jnp.maximum(_dot(h, dw1[...]) + db1[...], 0.0)
    out = jax.nn.sigmoid(_dot(h, dw2[...]) + db2[...])
    return out, jnp.stack(idx_rows, axis=0), jnp.stack(loss_sums)


def _rqvae_kernel(x_ref, ew0, eb0, ew1, eb1, ew2, eb2, cbT_ref, c2_ref,
                  dw0, db0, dw1, db1, dw2, db2,
                  out_ref, idx_ref, loss_ref):
    i = pl.program_id(0)
    ws = (ew0, eb0, ew1, eb1, ew2, eb2, cbT_ref, c2_ref,
          dw0, db0, dw1, db1, dw2, db2)
    out, idxs, loss = _forward(x_ref[...], *ws)
    out_ref[...] = out
    idx_ref[...] = idxs

    @pl.when(i == 0)
    def _():
        loss_ref[...] = jnp.zeros_like(loss_ref)
    loss_ref[...] += loss[None, :]


@jax.jit
def _run(x, enc_W0, enc_b0, enc_W1, enc_b1, enc_W2, enc_b2,
         codebooks, dec_W0, dec_b0, dec_W1, dec_b1, dec_W2, dec_b2):
    grid = BATCH // TILE
    full = lambda shape: pl.BlockSpec(shape, lambda i: (0,) * len(shape))
    cbT = codebooks.transpose(0, 2, 1)
    c2 = jnp.sum(codebooks ** 2, axis=2)[..., None]
    out, idxs, loss = pl.pallas_call(
        _rqvae_kernel,
        grid=(grid,),
        in_specs=[
            pl.BlockSpec((TILE, IN_DIM), lambda i: (i, 0)),
            full(enc_W0.shape), full((1, enc_b0.shape[0])),
            full(enc_W1.shape), full((1, enc_b1.shape[0])),
            full(enc_W2.shape), full((enc_b2.shape[0], 1)),
            full(cbT.shape), full(c2.shape),
            full(dec_W0.shape), full((1, dec_b0.shape[0])),
            full(dec_W1.shape), full((1, dec_b1.shape[0])),
            full(dec_W2.shape), full((1, dec_b2.shape[0])),
        ],
        out_specs=[
            pl.BlockSpec((TILE, IN_DIM), lambda i: (i, 0)),
            pl.BlockSpec((NUM_LEVELS, TILE), lambda i: (0, i)),
            pl.BlockSpec((1, NUM_LEVELS), lambda i: (0, 0)),
        ],
        out_shape=[
            jax.ShapeDtypeStruct((BATCH, IN_DIM), jnp.float32),
            jax.ShapeDtypeStruct((NUM_LEVELS, BATCH), jnp.int32),
            jax.ShapeDtypeStruct((1, NUM_LEVELS), jnp.float32),
        ],
    )(x, enc_W0, enc_b0.reshape(1, -1), enc_W1, enc_b1.reshape(1, -1),
      enc_W2, enc_b2.reshape(-1, 1), cbT, c2,
      dec_W0, dec_b0.reshape(1, -1), dec_W1, dec_b1.reshape(1, -1),
      dec_W2, dec_b2.reshape(1, -1))
    per_level_mse = loss[0] / (BATCH * E_DIM)
    rq_loss = jnp.mean((1.0 + BETA) * per_level_mse)
    return out, rq_loss, idxs.T


def kernel(x, epoch_idx, enc_W0, enc_b0, enc_W1, enc_b1, enc_W2, enc_b2,
           codebooks, dec_W0, dec_b0, dec_W1, dec_b1, dec_W2, dec_b2):
    return _run(x, enc_W0, enc_b0, enc_W1, enc_b1, enc_W2, enc_b2,
                codebooks, dec_W0, dec_b0, dec_W1, dec_b1, dec_W2, dec_b2)


# speedup vs baseline: 1.3572x; 1.0337x over previous
import jax
import jax.numpy as jnp
from jax.experimental import pallas as pl

IN_DIM = 768
E_DIM = 64
NUM_LEVELS = 3
K = 1024
BETA = 0.25
BATCH = 16384
TILE = 512
HALF = TILE // 2
_CHUNK = 128

_DN = lambda lc, rc: ((lc, rc), ((), ()))


def _dot(a, b, dims=(((1,), (0,)), ((), ()))):
    return jax.lax.dot_general(a, b, dims,
                               precision=jax.lax.Precision.DEFAULT,
                               preferred_element_type=jnp.float32)


def _gather_rows(cbT, idx, n):
    lo = jnp.bitwise_and(idx, _CHUNK - 1)
    hi = jnp.right_shift(idx, 7)
    lo_b = jax.lax.broadcast_in_dim(lo, (E_DIM, n), (1,))
    hi_b = jax.lax.broadcast_in_dim(hi, (E_DIM, n), (1,))
    xqT = jnp.zeros((E_DIM, n), jnp.float32)
    for h in range(K // _CHUNK):
        g = jnp.take_along_axis(cbT[:, h * _CHUNK:(h + 1) * _CHUNK], lo_b,
                                axis=1)
        xqT = jnp.where(hi_b == h, g, xqT)
    return xqT


def _forward(x, ew0, eb0, ew1, eb1, ew2, eb2, cbT_ref, c2_ref,
             dw0, db0, dw1, db1, dw2, db2):
    n = x.shape[0]
    h = jnp.maximum(_dot(x, ew0[...]) + eb0[...], 0.0)
    h = jnp.maximum(_dot(h, ew1[...]) + eb1[...], 0.0)
    resT = _dot(ew2[...], h, _DN((0,), (1,))) + eb2[...]

    kiota = jax.lax.broadcasted_iota(jnp.int32, (K, n), 0)
    xq_accT = jnp.zeros_like(resT)
    loss_sums = []
    idx_rows = []
    for lvl in range(NUM_LEVELS):
        cbT = cbT_ref[lvl]
        c2 = jnp.sum(cbT * cbT, axis=0)[:, None]
        d = c2 - 2.0 * _dot(cbT, resT, _DN((0,), (0,)))
        m = jnp.min(d, axis=0, keepdims=True)
        idx = jnp.min(jnp.where(d == m, kiota, K), axis=0)
        xqT = _gather_rows(cbT, idx, n)
        diffT = xqT - resT
        loss_sums.append(jnp.sum(diffT * diffT))
        xq_accT = xq_accT + xqT
        resT = resT - xqT
        idx_rows.append(idx)

    h = jnp.maximum(_dot(xq_accT, dw0[...], _DN((0,), (0,))) + db0[...], 0.0)
    h = jnp.maximum(_dot(h, dw1[...]) + db1[...], 0.0)
    out = jax.nn.sigmoid(_dot(h, dw2[...]) + db2[...])
    return out, jnp.stack(idx_rows, axis=0), jnp.stack(loss_sums)


def _rqvae_kernel(x_ref, ew0, eb0, ew1, eb1, ew2, eb2, cbT_ref, c2_ref,
                  dw0, db0, dw1, db1, dw2, db2,
                  out_ref, idx_ref, loss_ref):
    i = pl.program_id(0)
    ws = (ew0, eb0, ew1, eb1, ew2, eb2, cbT_ref, c2_ref,
          dw0, db0, dw1, db1, dw2, db2)
    out, idxs, loss = _forward(x_ref[...], *ws)
    out_ref[...] = out
    idx_ref[...] = idxs

    @pl.when(i == 0)
    def _():
        loss_ref[...] = jnp.zeros_like(loss_ref)
    loss_ref[...] += loss[None, :]


@jax.jit
def _run(x, enc_W0, enc_b0, enc_W1, enc_b1, enc_W2, enc_b2,
         codebooks, dec_W0, dec_b0, dec_W1, dec_b1, dec_W2, dec_b2):
    grid = BATCH // TILE
    full = lambda shape: pl.BlockSpec(shape, lambda i: (0,) * len(shape))
    cbT = codebooks.transpose(0, 2, 1)
    c2 = jnp.sum(codebooks ** 2, axis=2)[..., None]
    out, idxs, loss = pl.pallas_call(
        _rqvae_kernel,
        grid=(grid,),
        in_specs=[
            pl.BlockSpec((TILE, IN_DIM), lambda i: (i, 0)),
            full(enc_W0.shape), full((1, enc_b0.shape[0])),
            full(enc_W1.shape), full((1, enc_b1.shape[0])),
            full(enc_W2.shape), full((enc_b2.shape[0], 1)),
            full(cbT.shape), full(c2.shape),
            full(dec_W0.shape), full((1, dec_b0.shape[0])),
            full(dec_W1.shape), full((1, dec_b1.shape[0])),
            full(dec_W2.shape), full((1, dec_b2.shape[0])),
        ],
        out_specs=[
            pl.BlockSpec((TILE, IN_DIM), lambda i: (i, 0)),
            pl.BlockSpec((NUM_LEVELS, TILE), lambda i: (0, i)),
            pl.BlockSpec((1, NUM_LEVELS), lambda i: (0, 0)),
        ],
        out_shape=[
            jax.ShapeDtypeStruct((BATCH, IN_DIM), jnp.float32),
            jax.ShapeDtypeStruct((NUM_LEVELS, BATCH), jnp.int32),
            jax.ShapeDtypeStruct((1, NUM_LEVELS), jnp.float32),
        ],
    )(x, enc_W0, enc_b0.reshape(1, -1), enc_W1, enc_b1.reshape(1, -1),
      enc_W2, enc_b2.reshape(-1, 1), cbT, c2,
      dec_W0, dec_b0.reshape(1, -1), dec_W1, dec_b1.reshape(1, -1),
      dec_W2, dec_b2.reshape(1, -1))
    per_level_mse = loss[0] / (BATCH * E_DIM)
    rq_loss = jnp.mean((1.0 + BETA) * per_level_mse)
    return out, rq_loss, idxs.T


def kernel(x, epoch_idx, enc_W0, enc_b0, enc_W1, enc_b1, enc_W2, enc_b2,
           codebooks, dec_W0, dec_b0, dec_W1, dec_b1, dec_W2, dec_b2):
    return _run(x, enc_W0, enc_b0, enc_W1, enc_b1, enc_W2, enc_b2,
                codebooks, dec_W0, dec_b0, dec_W1, dec_b1, dec_W2, dec_b2)


# speedup vs baseline: 1.8222x; 1.3426x over previous
import jax
import jax.numpy as jnp
from jax.experimental import pallas as pl

IN_DIM = 768
E_DIM = 64
NUM_LEVELS = 3
K = 1024
BETA = 0.25
BATCH = 16384
TILE = 512
NSTREAM = 2
BLOCK = TILE * NSTREAM
_CHUNK = 128

_DN = lambda lc, rc: ((lc, rc), ((), ()))


def _dot(a, b, dims=(((1,), (0,)), ((), ()))):
    return jax.lax.dot_general(a, b, dims,
                               precision=jax.lax.Precision.DEFAULT,
                               preferred_element_type=jnp.float32)


def _gather_rows(cbT, idx):
    lo = jnp.bitwise_and(idx, _CHUNK - 1)
    hi = jnp.right_shift(idx, 7)
    lo_b = jax.lax.broadcast_in_dim(lo, (E_DIM, TILE), (1,))
    hi_b = jax.lax.broadcast_in_dim(hi, (E_DIM, TILE), (1,))
    xqT = jnp.zeros((E_DIM, TILE), jnp.float32)
    for h in range(K // _CHUNK):
        g = jnp.take_along_axis(cbT[:, h * _CHUNK:(h + 1) * _CHUNK], lo_b,
                                axis=1)
        xqT = jnp.where(hi_b == h, g, xqT)
    return xqT


def _rqvae_kernel(x_ref, ew0, eb0, ew1, eb1, ew2, eb2, cbT_ref,
                  dw0, db0, dw1, db1, dw2, db2,
                  out_ref, idx_ref, loss_ref):
    i = pl.program_id(0)
    xs = [x_ref[s * TILE:(s + 1) * TILE] for s in range(NSTREAM)]
    hs = [jnp.maximum(_dot(x, ew0[...]) + eb0[...], 0.0) for x in xs]
    hs = [jnp.maximum(_dot(h, ew1[...]) + eb1[...], 0.0) for h in hs]
    rs = [_dot(ew2[...], h, _DN((0,), (1,))) + eb2[...] for h in hs]

    kiota = jax.lax.broadcasted_iota(jnp.int32, (K, TILE), 0)
    accs = [jnp.zeros((E_DIM, TILE), jnp.float32) for _ in range(NSTREAM)]
    loss_sums = []
    idx_rows = [[] for _ in range(NSTREAM)]
    for lvl in range(NUM_LEVELS):
        cbT = cbT_ref[lvl]
        c2 = jnp.sum(cbT * cbT, axis=0)[:, None]
        ds = [c2 - 2.0 * _dot(cbT, r, _DN((0,), (0,))) for r in rs]
        ms = [jnp.min(d, axis=0, keepdims=True) for d in ds]
        idxs = [jnp.min(jnp.where(d == m, kiota, K), axis=0)
                for d, m in zip(ds, ms)]
        xqs = [_gather_rows(cbT, idx) for idx in idxs]
        diffs = [xq - r for xq, r in zip(xqs, rs)]
        loss_sums.append(sum(jnp.sum(df * df) for df in diffs))
        accs = [a + xq for a, xq in zip(accs, xqs)]
        rs = [r - xq for r, xq in zip(rs, xqs)]
        for s in range(NSTREAM):
            idx_rows[s].append(idxs[s])

    hs = [jnp.maximum(_dot(a, dw0[...], _DN((0,), (0,))) + db0[...], 0.0)
          for a in accs]
    hs = [jnp.maximum(_dot(h, dw1[...]) + db1[...], 0.0) for h in hs]
    outs = [jax.nn.sigmoid(_dot(h, dw2[...]) + db2[...]) for h in hs]
    for s in range(NSTREAM):
        out_ref[s * TILE:(s + 1) * TILE] = outs[s]
        idx_ref[:, s * TILE:(s + 1) * TILE] = jnp.stack(idx_rows[s], axis=0)

    @pl.when(i == 0)
    def _():
        loss_ref[...] = jnp.zeros_like(loss_ref)
    loss_ref[...] += jnp.stack(loss_sums)[None, :]


@jax.jit
def _run(x, enc_W0, enc_b0, enc_W1, enc_b1, enc_W2, enc_b2,
         codebooks, dec_W0, dec_b0, dec_W1, dec_b1, dec_W2, dec_b2):
    grid = BATCH // BLOCK
    full = lambda shape: pl.BlockSpec(shape, lambda i: (0,) * len(shape))
    cbT = codebooks.transpose(0, 2, 1)
    out, idxs, loss = pl.pallas_call(
        _rqvae_kernel,
        grid=(grid,),
        in_specs=[
            pl.BlockSpec((BLOCK, IN_DIM), lambda i: (i, 0)),
            full(enc_W0.shape), full((1, enc_b0.shape[0])),
            full(enc_W1.shape), full((1, enc_b1.shape[0])),
            full(enc_W2.shape), full((enc_b2.shape[0], 1)),
            full(cbT.shape),
            full(dec_W0.shape), full((1, dec_b0.shape[0])),
            full(dec_W1.shape), full((1, dec_b1.shape[0])),
            full(dec_W2.shape), full((1, dec_b2.shape[0])),
        ],
        out_specs=[
            pl.BlockSpec((BLOCK, IN_DIM), lambda i: (i, 0)),
            pl.BlockSpec((NUM_LEVELS, BLOCK), lambda i: (0, i)),
            pl.BlockSpec((1, NUM_LEVELS), lambda i: (0, 0)),
        ],
        out_shape=[
            jax.ShapeDtypeStruct((BATCH, IN_DIM), jnp.float32),
            jax.ShapeDtypeStruct((NUM_LEVELS, BATCH), jnp.int32),
            jax.ShapeDtypeStruct((1, NUM_LEVELS), jnp.float32),
        ],
    )(x, enc_W0, enc_b0.reshape(1, -1), enc_W1, enc_b1.reshape(1, -1),
      enc_W2, enc_b2.reshape(-1, 1), cbT,
      dec_W0, dec_b0.reshape(1, -1), dec_W1, dec_b1.reshape(1, -1),
      dec_W2, dec_b2.reshape(1, -1))
    per_level_mse = loss[0] / (BATCH * E_DIM)
    rq_loss = jnp.mean((1.0 + BETA) * per_level_mse)
    return out, rq_loss, idxs.T


def kernel(x, epoch_idx, enc_W0, enc_b0, enc_W1, enc_b1, enc_W2, enc_b2,
           codebooks, dec_W0, dec_b0, dec_W1, dec_b1, dec_W2, dec_b2):
    return _run(x, enc_W0, enc_b0, enc_W1, enc_b1, enc_W2, enc_b2,
                codebooks, dec_W0, dec_b0, dec_W1, dec_b1, dec_W2, dec_b2)


# speedup vs baseline: 2.1060x; 1.1557x over previous
import jax
import jax.numpy as jnp
from jax.experimental import pallas as pl

IN_DIM = 768
E_DIM = 64
NUM_LEVELS = 3
K = 1024
BETA = 0.25
BATCH = 16384
TILE = 512
NSTREAM = 2
BLOCK = TILE * NSTREAM
_CHUNK = 128

_DN = lambda lc, rc: ((lc, rc), ((), ()))


def _dot(a, b, dims=(((1,), (0,)), ((), ()))):
    return jax.lax.dot_general(a, b, dims,
                               precision=jax.lax.Precision.DEFAULT,
                               preferred_element_type=jnp.float32)


def _argmin_sublanes(d):
    vals = [d[8 * h:8 * (h + 1)] for h in range(K // 8)]
    idxs = None
    while len(vals) > 1:
        if idxs is None:
            nv, ni = [], []
            for j in range(0, len(vals), 2):
                a, b = vals[j], vals[j + 1]
                mask = b < a
                nv.append(jnp.where(mask, b, a))
                ni.append(jnp.where(mask, jnp.int32(j + 1), jnp.int32(j)))
            vals, idxs = nv, ni
        else:
            nv, ni = [], []
            for j in range(0, len(vals), 2):
                mask = vals[j + 1] < vals[j]
                nv.append(jnp.where(mask, vals[j + 1], vals[j]))
                ni.append(jnp.where(mask, idxs[j + 1], idxs[j]))
            vals, idxs = nv, ni
    val, idx = vals[0], idxs[0]
    r = idx * 8 + jax.lax.broadcasted_iota(jnp.int32, val.shape, 0)
    m8 = jnp.min(val, axis=0, keepdims=True)
    return jnp.min(jnp.where(val == m8, r, K), axis=0)


def _gather_rows(cbT, idx):
    lo = jnp.bitwise_and(idx, _CHUNK - 1)
    hi = jnp.right_shift(idx, 7)
    lo_b = jax.lax.broadcast_in_dim(lo, (E_DIM, TILE), (1,))
    hi_b = jax.lax.broadcast_in_dim(hi, (E_DIM, TILE), (1,))
    xqT = jnp.zeros((E_DIM, TILE), jnp.float32)
    for h in range(K // _CHUNK):
        g = jnp.take_along_axis(cbT[:, h * _CHUNK:(h + 1) * _CHUNK], lo_b,
                                axis=1)
        xqT = jnp.where(hi_b == h, g, xqT)
    return xqT


def _rqvae_kernel(x_ref, ew0, eb0, ew1, eb1, ew2, eb2, cbT_ref, cbTm2_ref,
                  dw0, db0, dw1, db1, dw2, db2,
                  out_ref, idx_ref, loss_ref):
    i = pl.program_id(0)
    xs = [x_ref[s * TILE:(s + 1) * TILE] for s in range(NSTREAM)]
    hs = [jnp.maximum(_dot(x, ew0[...]) + eb0[...], 0.0) for x in xs]
    hs = [jnp.maximum(_dot(h, ew1[...]) + eb1[...], 0.0) for h in hs]
    rs = [_dot(ew2[...], h, _DN((0,), (1,))) + eb2[...] for h in hs]

    accs = [jnp.zeros((E_DIM, TILE), jnp.float32) for _ in range(NSTREAM)]
    loss_sums = []
    idx_rows = [[] for _ in range(NSTREAM)]
    for lvl in range(NUM_LEVELS):
        cbT = cbT_ref[lvl]
        c2 = jnp.sum(cbT * cbT, axis=0)[:, None]
        ds = [c2 + _dot(cbTm2_ref[lvl], r, _DN((0,), (0,))) for r in rs]
        idxs = [_argmin_sublanes(d) for d in ds]
        xqs = [_gather_rows(cbT, idx) for idx in idxs]
        diffs = [xq - r for xq, r in zip(xqs, rs)]
        loss_sums.append(sum(jnp.sum(df * df) for df in diffs))
        accs = [a + xq for a, xq in zip(accs, xqs)]
        rs = [r - xq for r, xq in zip(rs, xqs)]
        for s in range(NSTREAM):
            idx_rows[s].append(idxs[s])

    hs = [jnp.maximum(_dot(a, dw0[...], _DN((0,), (0,))) + db0[...], 0.0)
          for a in accs]
    hs = [jnp.maximum(_dot(h, dw1[...]) + db1[...], 0.0) for h in hs]
    outs = [jax.nn.sigmoid(_dot(h, dw2[...]) + db2[...]) for h in hs]
    for s in range(NSTREAM):
        out_ref[s * TILE:(s + 1) * TILE] = outs[s]
        idx_ref[:, s * TILE:(s + 1) * TILE] = jnp.stack(idx_rows[s], axis=0)

    @pl.when(i == 0)
    def _():
        loss_ref[...] = jnp.zeros_like(loss_ref)
    loss_ref[...] += jnp.stack(loss_sums)[None, :]


@jax.jit
def _run(x, enc_W0, enc_b0, enc_W1, enc_b1, enc_W2, enc_b2,
         codebooks, dec_W0, dec_b0, dec_W1, dec_b1, dec_W2, dec_b2):
    grid = BATCH // BLOCK
    full = lambda shape: pl.BlockSpec(shape, lambda i: (0,) * len(shape))
    cbT = codebooks.transpose(0, 2, 1)
    cbTm2 = -2.0 * cbT
    out, idxs, loss = pl.pallas_call(
        _rqvae_kernel,
        grid=(grid,),
        in_specs=[
            pl.BlockSpec((BLOCK, IN_DIM), lambda i: (i, 0)),
            full(enc_W0.shape), full((1, enc_b0.shape[0])),
            full(enc_W1.shape), full((1, enc_b1.shape[0])),
            full(enc_W2.shape), full((enc_b2.shape[0], 1)),
            full(cbT.shape), full(cbTm2.shape),
            full(dec_W0.shape), full((1, dec_b0.shape[0])),
            full(dec_W1.shape), full((1, dec_b1.shape[0])),
            full(dec_W2.shape), full((1, dec_b2.shape[0])),
        ],
        out_specs=[
            pl.BlockSpec((BLOCK, IN_DIM), lambda i: (i, 0)),
            pl.BlockSpec((NUM_LEVELS, BLOCK), lambda i: (0, i)),
            pl.BlockSpec((1, NUM_LEVELS), lambda i: (0, 0)),
        ],
        out_shape=[
            jax.ShapeDtypeStruct((BATCH, IN_DIM), jnp.float32),
            jax.ShapeDtypeStruct((NUM_LEVELS, BATCH), jnp.int32),
            jax.ShapeDtypeStruct((1, NUM_LEVELS), jnp.float32),
        ],
    )(x, enc_W0, enc_b0.reshape(1, -1), enc_W1, enc_b1.reshape(1, -1),
      enc_W2, enc_b2.reshape(-1, 1), cbT, cbTm2,
      dec_W0, dec_b0.reshape(1, -1), dec_W1, dec_b1.reshape(1, -1),
      dec_W2, dec_b2.reshape(1, -1))
    per_level_mse = loss[0] / (BATCH * E_DIM)
    rq_loss = jnp.mean((1.0 + BETA) * per_level_mse)
    return out, rq_loss, idxs.T


def kernel(x, epoch_idx, enc_W0, enc_b0, enc_W1, enc_b1, enc_W2, enc_b2,
           codebooks, dec_W0, dec_b0, dec_W1, dec_b1, dec_W2, dec_b2):
    return _run(x, enc_W0, enc_b0, enc_W1, enc_b1, enc_W2, enc_b2,
                codebooks, dec_W0, dec_b0, dec_W1, dec_b1, dec_W2, dec_b2)


# speedup vs baseline: 2.1109x; 1.0023x over previous
import jax
import jax.numpy as jnp
from jax.experimental import pallas as pl

IN_DIM = 768
E_DIM = 64
NUM_LEVELS = 3
K = 1024
BETA = 0.25
BATCH = 16384
TILE = 512
NSTREAM = 2
BLOCK = TILE * NSTREAM
_CHUNK = 128

_DN = lambda lc, rc: ((lc, rc), ((), ()))


def _dot(a, b, dims=(((1,), (0,)), ((), ()))):
    return jax.lax.dot_general(a, b, dims,
                               precision=jax.lax.Precision.DEFAULT,
                               preferred_element_type=jnp.float32)


def _argmin_sublanes(d):
    vals = [d[8 * h:8 * (h + 1)] for h in range(K // 8)]
    idxs = None
    while len(vals) > 1:
        if idxs is None:
            nv, ni = [], []
            for j in range(0, len(vals), 2):
                a, b = vals[j], vals[j + 1]
                mask = b < a
                nv.append(jnp.where(mask, b, a))
                ni.append(jnp.where(mask, jnp.int32(j + 1), jnp.int32(j)))
            vals, idxs = nv, ni
        else:
            nv, ni = [], []
            for j in range(0, len(vals), 2):
                mask = vals[j + 1] < vals[j]
                nv.append(jnp.where(mask, vals[j + 1], vals[j]))
                ni.append(jnp.where(mask, idxs[j + 1], idxs[j]))
            vals, idxs = nv, ni
    val, idx = vals[0], idxs[0]
    r = idx * 8 + jax.lax.broadcasted_iota(jnp.int32, val.shape, 0)
    m8 = jnp.min(val, axis=0, keepdims=True)
    return jnp.min(jnp.where(val == m8, r, K), axis=0)


def _gather_rows(cbT, idx):
    lo = jnp.bitwise_and(idx, _CHUNK - 1)
    hi = jnp.right_shift(idx, 7)
    lo_b = jax.lax.broadcast_in_dim(lo, (E_DIM, TILE), (1,))
    hi_b = jax.lax.broadcast_in_dim(hi, (E_DIM, TILE), (1,))
    xqT = jnp.zeros((E_DIM, TILE), jnp.float32)
    for h in range(K // _CHUNK):
        g = jnp.take_along_axis(cbT[:, h * _CHUNK:(h + 1) * _CHUNK], lo_b,
                                axis=1)
        xqT = jnp.where(hi_b == h, g, xqT)
    return xqT


def _rqvae_kernel(x_ref, ew0, eb0, ew1, eb1, ew2, eb2, cbT_ref, cbTm2_ref,
                  dw0, db0, dw1, db1, dw2, db2,
                  out_ref, idx_ref, loss_ref):
    i = pl.program_id(0)
    h = jnp.maximum(_dot(x_ref[...], ew0[...]) + eb0[...], 0.0)
    h = jnp.maximum(_dot(h, ew1[...]) + eb1[...], 0.0)
    resT_all = _dot(ew2[...], h, _DN((0,), (1,))) + eb2[...]
    rs = [resT_all[:, s * TILE:(s + 1) * TILE] for s in range(NSTREAM)]

    accs = [jnp.zeros((E_DIM, TILE), jnp.float32) for _ in range(NSTREAM)]
    loss_sums = []
    idx_rows = [[] for _ in range(NSTREAM)]
    for lvl in range(NUM_LEVELS):
        cbT = cbT_ref[lvl]
        c2 = jnp.sum(cbT * cbT, axis=0)[:, None]
        ds = [c2 + _dot(cbTm2_ref[lvl], r, _DN((0,), (0,))) for r in rs]
        idxs = [_argmin_sublanes(d) for d in ds]
        xqs = [_gather_rows(cbT, idx) for idx in idxs]
        diffs = [xq - r for xq, r in zip(xqs, rs)]
        loss_sums.append(sum(jnp.sum(df * df) for df in diffs))
        accs = [a + xq for a, xq in zip(accs, xqs)]
        rs = [r - xq for r, xq in zip(rs, xqs)]
        for s in range(NSTREAM):
            idx_rows[s].append(idxs[s])

    acc_all = jnp.concatenate(accs, axis=1)
    h = jnp.maximum(_dot(acc_all, dw0[...], _DN((0,), (0,))) + db0[...], 0.0)
    h = jnp.maximum(_dot(h, dw1[...]) + db1[...], 0.0)
    out_ref[...] = jax.nn.sigmoid(_dot(h, dw2[...]) + db2[...])
    for s in range(NSTREAM):
        idx_ref[:, s * TILE:(s + 1) * TILE] = jnp.stack(idx_rows[s], axis=0)

    @pl.when(i == 0)
    def _():
        loss_ref[...] = jnp.zeros_like(loss_ref)
    loss_ref[...] += jnp.stack(loss_sums)[None, :]


@jax.jit
def _run(x, enc_W0, enc_b0, enc_W1, enc_b1, enc_W2, enc_b2,
         codebooks, dec_W0, dec_b0, dec_W1, dec_b1, dec_W2, dec_b2):
    grid = BATCH // BLOCK
    full = lambda shape: pl.BlockSpec(shape, lambda i: (0,) * len(shape))
    cbT = codebooks.transpose(0, 2, 1)
    cbTm2 = -2.0 * cbT
    out, idxs, loss = pl.pallas_call(
        _rqvae_kernel,
        grid=(grid,),
        in_specs=[
            pl.BlockSpec((BLOCK, IN_DIM), lambda i: (i, 0)),
            full(enc_W0.shape), full((1, enc_b0.shape[0])),
            full(enc_W1.shape), full((1, enc_b1.shape[0])),
            full(enc_W2.shape), full((enc_b2.shape[0], 1)),
            full(cbT.shape), full(cbTm2.shape),
            full(dec_W0.shape), full((1, dec_b0.shape[0])),
            full(dec_W1.shape), full((1, dec_b1.shape[0])),
            full(dec_W2.shape), full((1, dec_b2.shape[0])),
        ],
        out_specs=[
            pl.BlockSpec((BLOCK, IN_DIM), lambda i: (i, 0)),
            pl.BlockSpec((NUM_LEVELS, BLOCK), lambda i: (0, i)),
            pl.BlockSpec((1, NUM_LEVELS), lambda i: (0, 0)),
        ],
        out_shape=[
            jax.ShapeDtypeStruct((BATCH, IN_DIM), jnp.float32),
            jax.ShapeDtypeStruct((NUM_LEVELS, BATCH), jnp.int32),
            jax.ShapeDtypeStruct((1, NUM_LEVELS), jnp.float32),
        ],
    )(x, enc_W0, enc_b0.reshape(1, -1), enc_W1, enc_b1.reshape(1, -1),
      enc_W2, enc_b2.reshape(-1, 1), cbT, cbTm2,
      dec_W0, dec_b0.reshape(1, -1), dec_W1, dec_b1.reshape(1, -1),
      dec_W2, dec_b2.reshape(1, -1))
    per_level_mse = loss[0] / (BATCH * E_DIM)
    rq_loss = jnp.mean((1.0 + BETA) * per_level_mse)
    return out, rq_loss, idxs.T


def kernel(x, epoch_idx, enc_W0, enc_b0, enc_W1, enc_b1, enc_W2, enc_b2,
           codebooks, dec_W0, dec_b0, dec_W1, dec_b1, dec_W2, dec_b2):
    return _run(x, enc_W0, enc_b0, enc_W1, enc_b1, enc_W2, enc_b2,
                codebooks, dec_W0, dec_b0, dec_W1, dec_b1, dec_W2, dec_b2)
